# baseline (speedup 1.0000x reference)
import functools

import jax
import jax.numpy as jnp
from jax import lax
from jax.experimental import pallas as pl
from jax.experimental.pallas import tpu as pltpu


def _to_key(x_f32, tag):
    b = lax.bitcast_convert_type(x_f32, jnp.int32)
    key = b ^ ((b >> 31) & jnp.int32(0x7FFFFFFF))
    return (key & jnp.int32(-2)) | tag


def _from_key(key):
    b = key ^ ((key >> 31) & jnp.int32(0x7FFFFFFF))
    return lax.bitcast_convert_type(b, jnp.float32)


def _loss_body(t1_ref, t2_ref, out_ref, key_ref, s_ref, kb_ref, *, R, C, LR, LC, CH):
    n = (R * C) // 2
    logn = LR + LC
    LCH = CH.bit_length() - 1
    NCH = R // CH
    rloc = lax.broadcasted_iota(jnp.int32, (CH, C), 0)
    cloc = lax.broadcasted_iota(jnp.int32, (CH, C), 1)

    def idx_arr(start):
        return cloc * R + (rloc + start)

    def bit0(a, b):
        return ((a >> b) & 1) == 0

    def init_chunk(c, _):
        st = c * CH
        x1 = t1_ref[0, pl.ds(st, CH), :]
        x2 = t2_ref[0, pl.ds(st, CH), :]
        x = jnp.concatenate([x1, x2], axis=1)
        tag = jnp.where(cloc < (C // 2), jnp.int32(1), jnp.int32(0))
        key_ref[pl.ds(st, CH), :] = _to_key(x, tag)
        return 0

    lax.fori_loop(0, NCH, init_chunk, 0)

    base = (cloc << LR) + rloc

    def roll_pass(j, k, axis):
        if axis == 0:
            sh = 1 << j
            lower = (rloc & sh) == 0
        else:
            sh = 1 << (j - LR)
            lower = (cloc & sh) == 0

        def per_chunk(c, _):
            st = c * CH
            ch = key_ref[pl.ds(st, CH), :]
            kb = kb_ref[...]
            stbit = (st >> k) & 1
            down = jnp.roll(ch, -sh, axis=axis)
            up = jnp.roll(ch, sh, axis=axis)
            p = jnp.where(lower, down, up)
            keep_min = ((kb | stbit) == 0) == lower
            key_ref[pl.ds(st, CH), :] = jnp.where(
                (ch < p) == keep_min, ch, p
            )
            return 0

        lax.fori_loop(0, NCH, per_chunk, 0)

    def pair_pass(j, k):
        jm = j - LCH

        def per_pair(m, _):
            ca = ((m >> jm) << (jm + 1)) + (m & ((1 << jm) - 1))
            cb = ca + (1 << jm)
            sa = ca * CH
            sb = cb * CH
            a = key_ref[pl.ds(sa, CH), :]
            b = key_ref[pl.ds(sb, CH), :]
            asc = (kb_ref[...] | ((sa >> k) & 1)) == 0
            c1 = (a < b) == asc
            key_ref[pl.ds(sa, CH), :] = jnp.where(c1, a, b)
            key_ref[pl.ds(sb, CH), :] = jnp.where(c1, b, a)
            return 0

        lax.fori_loop(0, NCH // 2, per_pair, 0)

    def stage(k, _):
        kb_ref[...] = (base >> k) & 1
        for j in range(logn - 1, LR - 1, -1):
            @pl.when(j <= k - 1)
            def _(j=j):
                roll_pass(j, k, axis=1)

        jhi = jnp.minimum(k - 1, LR - 1)

        def glob_j(t, _):
            pair_pass(jhi - t, k)
            return 0

        lax.fori_loop(0, jnp.maximum(jhi - (LCH - 1), 0), glob_j, 0)

        for j in range(LCH - 1, -1, -1):
            @pl.when(j <= k - 1)
            def _(j=j):
                roll_pass(j, k, axis=0)
        return 0

    lax.fori_loop(1, logn + 1, stage, 0)

    def scan_chunk(c, carry):
        st = c * CH
        key = key_ref[pl.ds(st, CH), :]
        s = ((key & 1) * 2 - 1).astype(jnp.float32)
        for t in range(LCH):
            sh = 1 << t
            s = s + jnp.where(rloc >= sh, jnp.roll(s, sh, axis=0), 0.0)
        tot = s[CH - 1 : CH, :]
        s_ref[pl.ds(st, CH), :] = s + carry
        return carry + tot

    col_tot = lax.fori_loop(
        0, NCH, scan_chunk, jnp.zeros((1, C), jnp.float32)
    )
    ccol = lax.broadcasted_iota(jnp.int32, (1, C), 1)
    inc = col_tot
    for t in range(LC):
        sh = 1 << t
        inc = inc + jnp.where(ccol >= sh, jnp.roll(inc, sh, axis=1), 0.0)
    excl = jnp.where(ccol >= 1, jnp.roll(inc, 1, axis=1), 0.0)

    def tail_chunk(c, acc):
        st = c * CH
        v = _from_key(key_ref[pl.ds(st, CH), :])
        last = c == NCH - 1
        nstart = jnp.where(last, 0, st + CH)
        frow = _from_key(key_ref[pl.ds(nstart, 1), :])
        frow = jnp.where(last, jnp.roll(frow, -1, axis=1), frow)
        nv = jnp.where(rloc < CH - 1, pltpu.roll(v, CH - 1, axis=0), frow)
        delta = nv - v
        delta = jnp.where(
            last & (rloc == CH - 1) & (cloc == C - 1), 0.0, delta
        )
        d = s_ref[pl.ds(st, CH), :] + excl
        return acc + jnp.sum(d * d * delta)

    acc = lax.fori_loop(0, NCH, tail_chunk, jnp.float32(0.0))
    loss = jnp.sqrt(acc) / jnp.float32(n)
    out_ref[...] = jnp.broadcast_to(loss, (1, 1, 128))


@functools.partial(jax.jit, static_argnums=(2, 3))
def _wass_losses(t1, t2, R, C):
    B = t1.shape[0]
    LR = R.bit_length() - 1
    LC = C.bit_length() - 1
    assert (1 << LR) == R and (1 << LC) == C
    CH = min(R, 1024)
    t1r = t1.reshape(B, R, C // 2)
    t2r = t2.reshape(B, R, C // 2)
    body = functools.partial(_loss_body, R=R, C=C, LR=LR, LC=LC, CH=CH)
    out = pl.pallas_call(
        body,
        grid=(B,),
        in_specs=[
            pl.BlockSpec((1, R, C // 2), lambda i: (i, 0, 0)),
            pl.BlockSpec((1, R, C // 2), lambda i: (i, 0, 0)),
        ],
        out_specs=pl.BlockSpec((1, 1, 128), lambda i: (i, 0, 0)),
        out_shape=jax.ShapeDtypeStruct((B, 1, 128), jnp.float32),
        scratch_shapes=[
            pltpu.VMEM((R, C), jnp.int32),
            pltpu.VMEM((R, C), jnp.float32),
            pltpu.VMEM((CH, C), jnp.int32),
        ],
        compiler_params=pltpu.CompilerParams(
            dimension_semantics=("arbitrary",),
            vmem_limit_bytes=100 * 1024 * 1024,
        ),
    )(t1r, t2r)
    return out[:, 0, 0]


def kernel(t1, t2):
    B = t1.shape[0]
    n = t1.shape[1] * t1.shape[2]
    N = 2 * n
    C = 256 if N >= 2048 else max(2, N // 8)
    R = N // C
    losses = _wass_losses(t1, t2, R, C)
    return jnp.mean(losses)


# speedup vs baseline: 1.4322x; 1.4322x over previous
import functools

import jax
import jax.numpy as jnp
from jax import lax
from jax.experimental import pallas as pl
from jax.experimental.pallas import tpu as pltpu


def _to_key(x_f32, tag):
    b = lax.bitcast_convert_type(x_f32, jnp.int32)
    key = b ^ ((b >> 31) & jnp.int32(0x7FFFFFFF))
    return (key & jnp.int32(-2)) | tag


def _from_key(key):
    b = key ^ ((key >> 31) & jnp.int32(0x7FFFFFFF))
    return lax.bitcast_convert_type(b, jnp.float32)


def _loss_body(t1_ref, t2_ref, out_ref, key_ref, s_ref, kb_ref, *, R, C, LR, LC, CH):
    n = (R * C) // 2
    logn = LR + LC
    LCH = CH.bit_length() - 1
    NCH = R // CH
    rloc = lax.broadcasted_iota(jnp.int32, (CH, C), 0)
    cloc = lax.broadcasted_iota(jnp.int32, (CH, C), 1)

    def idx_arr(start):
        return cloc * R + (rloc + start)

    def bit0(a, b):
        return ((a >> b) & 1) == 0

    def init_chunk(c, _):
        st = c * CH
        x1 = t1_ref[0, pl.ds(st, CH), :]
        x2 = t2_ref[0, pl.ds(st, CH), :]
        x = jnp.concatenate([x1, x2], axis=1)
        tag = jnp.where(cloc < (C // 2), jnp.int32(1), jnp.int32(0))
        key_ref[pl.ds(st, CH), :] = _to_key(x, tag)
        return 0

    lax.fori_loop(0, NCH, init_chunk, 0)

    base = (cloc << LR) + rloc

    def roll_pass(j, k, axis):
        if axis == 0:
            sh = 1 << j
            lower = (rloc & sh) == 0
        else:
            sh = 1 << (j - LR)
            lower = (cloc & sh) == 0

        def per_chunk(c, _):
            st = c * CH
            ch = key_ref[pl.ds(st, CH), :]
            kb = kb_ref[...]
            stbit = (st >> k) & 1
            down = jnp.roll(ch, -sh, axis=axis)
            up = jnp.roll(ch, sh, axis=axis)
            p = jnp.where(lower, down, up)
            keep_min = ((kb | stbit) == 0) == lower
            key_ref[pl.ds(st, CH), :] = jnp.where(
                keep_min, jnp.minimum(ch, p), jnp.maximum(ch, p)
            )
            return 0

        lax.fori_loop(0, NCH, per_chunk, 0)

    def pair_pass(j, k):
        jm = j - LCH

        def per_pair(m, _):
            ca = ((m >> jm) << (jm + 1)) + (m & ((1 << jm) - 1))
            cb = ca + (1 << jm)
            sa = ca * CH
            sb = cb * CH
            a = key_ref[pl.ds(sa, CH), :]
            b = key_ref[pl.ds(sb, CH), :]
            asc = (kb_ref[...] | ((sa >> k) & 1)) == 0
            mn = jnp.minimum(a, b)
            mx = jnp.maximum(a, b)
            key_ref[pl.ds(sa, CH), :] = jnp.where(asc, mn, mx)
            key_ref[pl.ds(sb, CH), :] = jnp.where(asc, mx, mn)
            return 0

        lax.fori_loop(0, NCH // 2, per_pair, 0)

    def stage(k, _):
        kb_ref[...] = (base >> k) & 1
        for j in range(logn - 1, LR - 1, -1):
            @pl.when(j <= k - 1)
            def _(j=j):
                roll_pass(j, k, axis=1)

        jhi = jnp.minimum(k - 1, LR - 1)

        def glob_j(t, _):
            pair_pass(jhi - t, k)
            return 0

        lax.fori_loop(0, jnp.maximum(jhi - (LCH - 1), 0), glob_j, 0)

        for j in range(LCH - 1, -1, -1):
            @pl.when(j <= k - 1)
            def _(j=j):
                roll_pass(j, k, axis=0)
        return 0

    lax.fori_loop(1, logn + 1, stage, 0)

    def scan_chunk(c, carry):
        st = c * CH
        key = key_ref[pl.ds(st, CH), :]
        s = ((key & 1) * 2 - 1).astype(jnp.float32)
        for t in range(LCH):
            sh = 1 << t
            s = s + jnp.where(rloc >= sh, jnp.roll(s, sh, axis=0), 0.0)
        tot = s[CH - 1 : CH, :]
        s_ref[pl.ds(st, CH), :] = s + carry
        return carry + tot

    col_tot = lax.fori_loop(
        0, NCH, scan_chunk, jnp.zeros((1, C), jnp.float32)
    )
    ccol = lax.broadcasted_iota(jnp.int32, (1, C), 1)
    inc = col_tot
    for t in range(LC):
        sh = 1 << t
        inc = inc + jnp.where(ccol >= sh, jnp.roll(inc, sh, axis=1), 0.0)
    excl = jnp.where(ccol >= 1, jnp.roll(inc, 1, axis=1), 0.0)

    def tail_chunk(c, acc):
        st = c * CH
        v = _from_key(key_ref[pl.ds(st, CH), :])
        last = c == NCH - 1
        nstart = jnp.where(last, 0, st + CH)
        frow = _from_key(key_ref[pl.ds(nstart, 1), :])
        frow = jnp.where(last, jnp.roll(frow, -1, axis=1), frow)
        nv = jnp.where(rloc < CH - 1, pltpu.roll(v, CH - 1, axis=0), frow)
        delta = nv - v
        delta = jnp.where(
            last & (rloc == CH - 1) & (cloc == C - 1), 0.0, delta
        )
        d = s_ref[pl.ds(st, CH), :] + excl
        return acc + jnp.sum(d * d * delta)

    acc = lax.fori_loop(0, NCH, tail_chunk, jnp.float32(0.0))
    loss = jnp.sqrt(acc) / jnp.float32(n)
    out_ref[...] = jnp.broadcast_to(loss, (1, 1, 128))


@functools.partial(jax.jit, static_argnums=(2, 3))
def _wass_losses(t1, t2, R, C):
    B = t1.shape[0]
    LR = R.bit_length() - 1
    LC = C.bit_length() - 1
    assert (1 << LR) == R and (1 << LC) == C
    CH = min(R, 1024)
    t1r = t1.reshape(B, R, C // 2)
    t2r = t2.reshape(B, R, C // 2)
    body = functools.partial(_loss_body, R=R, C=C, LR=LR, LC=LC, CH=CH)
    out = pl.pallas_call(
        body,
        grid=(B,),
        in_specs=[
            pl.BlockSpec((1, R, C // 2), lambda i: (i, 0, 0)),
            pl.BlockSpec((1, R, C // 2), lambda i: (i, 0, 0)),
        ],
        out_specs=pl.BlockSpec((1, 1, 128), lambda i: (i, 0, 0)),
        out_shape=jax.ShapeDtypeStruct((B, 1, 128), jnp.float32),
        scratch_shapes=[
            pltpu.VMEM((R, C), jnp.int32),
            pltpu.VMEM((R, C), jnp.float32),
            pltpu.VMEM((CH, C), jnp.int32),
        ],
        compiler_params=pltpu.CompilerParams(
            dimension_semantics=("arbitrary",),
            vmem_limit_bytes=100 * 1024 * 1024,
        ),
    )(t1r, t2r)
    return out[:, 0, 0]


def kernel(t1, t2):
    B = t1.shape[0]
    n = t1.shape[1] * t1.shape[2]
    N = 2 * n
    C = 256 if N >= 2048 else max(2, N // 8)
    R = N // C
    losses = _wass_losses(t1, t2, R, C)
    return jnp.mean(losses)
